# baseline (speedup 1.0000x reference)
import functools

import jax
import jax.numpy as jnp
from jax import lax
from jax.experimental import pallas as pl
from jax.experimental.pallas import tpu as pltpu
from jax.experimental.pallas import tpu_sc as plsc

_V = 1000
_NQ = 400
_VB = 128
_NVB = (_V + _VB - 1) // _VB
_QPW = 16
_NWORK = _NQ // _QPW


def _dist_body(q_ref, e_ref, val_ref, idx_ref):
    q = q_ref[...]
    e = e_ref[...]
    e2 = jnp.sum(e * e, axis=1)
    qe = jax.lax.dot_general(
        e, q, (((1,), (1,)), ((), ())),
        preferred_element_type=jnp.float32,
        precision=jax.lax.Precision.HIGHEST,
    )
    dist = e2[:, None] - 2.0 * qe
    for b in range(_NVB):
        lo = b * _VB
        hi = min(lo + _VB, _V)
        blk = dist[lo:hi]
        val_ref[b, :] = jnp.min(blk, axis=0)
        idx_ref[b, :] = jnp.argmin(blk, axis=0).astype(jnp.int32) + lo


def _make_sc_merge():
    mesh = plsc.VectorSubcoreMesh(core_axis_name="c", subcore_axis_name="s")
    info = plsc.get_sparse_core_info()
    nc = info.num_cores

    @functools.partial(
        pl.kernel,
        out_type=jax.ShapeDtypeStruct((_NWORK, _QPW), jnp.int32),
        mesh=mesh,
        scratch_types=[
            pltpu.VMEM((_NVB, _QPW), jnp.float32),
            pltpu.VMEM((_NVB, _QPW), jnp.int32),
            pltpu.VMEM((_QPW,), jnp.int32),
        ],
        compiler_params=pltpu.CompilerParams(use_tc_tiling_on_sc=False),
    )
    def sc_merge(val_hbm, idx_hbm, out_hbm, val_v, idx_v, res_v):
        wid = lax.axis_index("s") * nc + lax.axis_index("c")

        @pl.when(wid < _NWORK)
        def _():
            col = pl.ds(wid * _QPW, _QPW)
            pltpu.sync_copy(val_hbm.at[:, col], val_v)
            pltpu.sync_copy(idx_hbm.at[:, col], idx_v)
            m = val_v[0]
            im = idx_v[0]
            for b in range(1, _NVB):
                x = val_v[b]
                pred = x < m
                m = jnp.where(pred, x, m)
                im = jnp.where(pred, idx_v[b], im)
            res_v[...] = im
            pltpu.sync_copy(res_v, out_hbm.at[wid])

    return sc_merge


def kernel(inputs, embeddings):
    B, S, D = inputs.shape
    q = inputs.reshape(B * S, D)
    val, idx = pl.pallas_call(
        _dist_body,
        out_shape=(
            jax.ShapeDtypeStruct((_NVB, _NQ), jnp.float32),
            jax.ShapeDtypeStruct((_NVB, _NQ), jnp.int32),
        ),
    )(q, embeddings)
    out = _make_sc_merge()(val, idx)
    return out.reshape(B, S)


# speedup vs baseline: 1.0493x; 1.0493x over previous
import functools

import jax
import jax.numpy as jnp
from jax import lax
from jax.experimental import pallas as pl
from jax.experimental.pallas import tpu as pltpu
from jax.experimental.pallas import tpu_sc as plsc

_V = 1000
_NQ = 400
_VB = 128
_NVB = (_V + _VB - 1) // _VB
_QPW = 16
_NWORK = _NQ // _QPW


def _dist_body(q_ref, e_ref, cand_ref):
    q = q_ref[...]
    e = e_ref[...]
    e2 = jnp.sum(e * e, axis=1)
    qe = jax.lax.dot_general(
        e, q, (((1,), (1,)), ((), ())),
        preferred_element_type=jnp.float32,
        precision=jax.lax.Precision.HIGHEST,
    )
    dist = e2[:, None] - 2.0 * qe
    for b in range(_NVB):
        lo = b * _VB
        hi = min(lo + _VB, _V)
        blk = dist[lo:hi]
        idx = jnp.argmin(blk, axis=0).astype(jnp.int32) + lo
        cand_ref[b, :] = jnp.min(blk, axis=0)
        cand_ref[_NVB + b, :] = jax.lax.bitcast_convert_type(idx, jnp.float32)


def _make_sc_merge():
    mesh = plsc.VectorSubcoreMesh(core_axis_name="c", subcore_axis_name="s")
    info = plsc.get_sparse_core_info()
    nc = info.num_cores

    @functools.partial(
        pl.kernel,
        out_type=jax.ShapeDtypeStruct((_NWORK, _QPW), jnp.int32),
        mesh=mesh,
        scratch_types=[
            pltpu.VMEM((2 * _NVB, _QPW), jnp.float32),
            pltpu.VMEM((_QPW,), jnp.int32),
        ],
        compiler_params=pltpu.CompilerParams(use_tc_tiling_on_sc=False),
    )
    def sc_merge(cand_hbm, out_hbm, cand_v, res_v):
        wid = lax.axis_index("s") * nc + lax.axis_index("c")

        @pl.when(wid < _NWORK)
        def _():
            col = pl.ds(wid * _QPW, _QPW)
            pltpu.sync_copy(cand_hbm.at[:, col], cand_v)
            m = cand_v[0]
            im = jax.lax.bitcast_convert_type(cand_v[_NVB], jnp.int32)
            for b in range(1, _NVB):
                x = cand_v[b]
                pred = x < m
                m = jnp.where(pred, x, m)
                im = jnp.where(pred, jax.lax.bitcast_convert_type(cand_v[_NVB + b], jnp.int32), im)
            res_v[...] = im
            pltpu.sync_copy(res_v, out_hbm.at[wid])

    return sc_merge


def kernel(inputs, embeddings):
    B, S, D = inputs.shape
    q = inputs.reshape(B * S, D)
    cand = pl.pallas_call(
        _dist_body,
        out_shape=jax.ShapeDtypeStruct((2 * _NVB, _NQ), jnp.float32),
    )(q, embeddings)
    out = _make_sc_merge()(cand)
    return out.reshape(B, S)


# speedup vs baseline: 1.0855x; 1.0345x over previous
import functools

import jax
import jax.numpy as jnp
from jax import lax
from jax.experimental import pallas as pl
from jax.experimental.pallas import tpu as pltpu
from jax.experimental.pallas import tpu_sc as plsc

_V = 1000
_NQ = 400
_VB = 128
_NVB = (_V + _VB - 1) // _VB
_QPW = 16
_NWORK = _NQ // _QPW
_NQP = 512


def _dist_body(q_ref, e_ref, cand_ref):
    q = q_ref[...]
    e = e_ref[...]
    e2 = jnp.sum(e * e, axis=1)
    qe = jax.lax.dot_general(
        e, q, (((1,), (1,)), ((), ())),
        preferred_element_type=jnp.float32,
        precision=jax.lax.Precision.HIGHEST,
    )
    dist = e2[:, None] - 2.0 * qe
    for b in range(_NVB):
        lo = b * _VB
        hi = min(lo + _VB, _V)
        blk = dist[lo:hi]
        idx = jnp.argmin(blk, axis=0).astype(jnp.int32) + lo
        cand_ref[b, : _NQ] = jnp.min(blk, axis=0)
        cand_ref[_NVB + b, : _NQ] = jax.lax.bitcast_convert_type(idx, jnp.float32)


def _make_sc_merge():
    mesh = plsc.VectorSubcoreMesh(core_axis_name="c", subcore_axis_name="s")
    info = plsc.get_sparse_core_info()
    nc = info.num_cores

    @functools.partial(
        pl.kernel,
        out_type=jax.ShapeDtypeStruct((_NWORK, _QPW), jnp.int32),
        mesh=mesh,
        scratch_types=[
            pltpu.VMEM((2 * _NVB, 128), jnp.float32),
            pltpu.VMEM((_QPW,), jnp.int32),
        ],
    )
    def sc_merge(cand_hbm, out_hbm, cand_v, res_v):
        wid = lax.axis_index("s") * nc + lax.axis_index("c")

        @pl.when(wid < _NWORK)
        def _():
            chunk = pl.ds((wid // 8) * 128, 128)
            sub = pl.ds((wid % 8) * _QPW, _QPW)
            pltpu.sync_copy(cand_hbm.at[:, chunk], cand_v)
            m = cand_v[0, sub]
            im = jax.lax.bitcast_convert_type(cand_v[_NVB, sub], jnp.int32)
            for b in range(1, _NVB):
                x = cand_v[b, sub]
                pred = x < m
                m = jnp.where(pred, x, m)
                im = jnp.where(pred, jax.lax.bitcast_convert_type(cand_v[_NVB + b, sub], jnp.int32), im)
            res_v[...] = im
            pltpu.sync_copy(res_v, out_hbm.at[wid])

    return sc_merge


def kernel(inputs, embeddings):
    B, S, D = inputs.shape
    q = inputs.reshape(B * S, D)
    cand = pl.pallas_call(
        _dist_body,
        out_shape=jax.ShapeDtypeStruct((2 * _NVB, _NQP), jnp.float32),
    )(q, embeddings)
    out = _make_sc_merge()(cand)
    return out.reshape(B, S)
